# baseline (speedup 1.0000x reference)
import functools

import jax
import jax.numpy as jnp
from jax import lax
from jax.experimental import pallas as pl
from jax.experimental.pallas import tpu as pltpu
from jax.experimental.pallas import tpu_sc as plsc

B, S, D = 16, 32, 1024
E, TOPK, DFF = 8, 2, 1024
T = B * S
BLK = 128
NR = T * TOPK
NPAD = NR + E * BLK


def _routing_body(x_ref, r_ref, rt_ref, rw_ref, nblk_ref, off_ref):
    x = x_ref[...]
    logits = lax.dot_general(x, r_ref[...], (((1,), (1,)), ((), ())),
                             preferred_element_type=jnp.float32)
    idx = lax.broadcasted_iota(jnp.int32, (T, E), 1)
    m0 = jnp.max(logits, axis=1, keepdims=True)
    i0 = jnp.min(jnp.where(logits == m0, idx, E), axis=1, keepdims=True)
    masked = jnp.where(idx == i0, -jnp.inf, logits)
    m1 = jnp.max(masked, axis=1, keepdims=True)
    i1 = jnp.min(jnp.where(masked == m1, idx, E), axis=1, keepdims=True)
    e1 = jnp.exp(m1 - m0)
    denom = 1.0 + e1
    w0 = 1.0 / denom
    w1 = e1 / denom
    s = w0 + w1
    w0 = w0 / s
    w1 = w1 / s

    tril = (lax.broadcasted_iota(jnp.int32, (T, T), 0)
            > lax.broadcasted_iota(jnp.int32, (T, T), 1)).astype(jnp.float32)
    pos0 = jnp.zeros((T, 1), jnp.float32)
    pos1 = jnp.zeros((T, 1), jnp.float32)
    offblk = jnp.int32(0)
    for e in range(E):
        m0e = (i0 == e)
        m1e = (i1 == e)
        f0 = m0e.astype(jnp.float32)
        f1 = m1e.astype(jnp.float32)
        cnt_t = f0 + f1
        pre = lax.dot_general(tril, cnt_t, (((1,), (0,)), ((), ())),
                              preferred_element_type=jnp.float32)
        count_e = jnp.sum(cnt_t).astype(jnp.int32)
        nb_e = (count_e + BLK - 1) // BLK
        base = (offblk * BLK).astype(jnp.float32)
        pos0 = pos0 + jnp.where(m0e, base + pre, 0.0)
        pos1 = pos1 + jnp.where(m1e, base + pre + f0, 0.0)
        nblk_ref[0, e] = nb_e
        off_ref[0, e] = offblk
        offblk = offblk + nb_e

    lane_r = lax.broadcasted_iota(jnp.int32, (T, NPAD), 1).astype(jnp.float32)
    m0t = jnp.where(lane_r == pos0, 1.0, 0.0)
    m1t = jnp.where(lane_r == pos1, 1.0, 0.0)
    t_col = lax.broadcasted_iota(jnp.int32, (T, 1), 0).astype(jnp.float32)
    rhs0 = jnp.concatenate([t_col, w0], axis=1)
    rhs1 = jnp.concatenate([t_col, w1], axis=1)
    out0 = lax.dot_general(m0t, rhs0, (((0,), (0,)), ((), ())),
                           preferred_element_type=jnp.float32)
    out1 = lax.dot_general(m1t, rhs1, (((0,), (0,)), ((), ())),
                           preferred_element_type=jnp.float32)
    rt_ref[...] = out0[:, 0:1] + out1[:, 0:1]
    rw_ref[...] = out0[:, 1:2] + out1[:, 1:2]


@jax.jit
def _routing(x, router):
    return pl.pallas_call(
        _routing_body,
        in_specs=[
            pl.BlockSpec((T, D), lambda: (0, 0)),
            pl.BlockSpec((E, D), lambda: (0, 0)),
        ],
        out_specs=[
            pl.BlockSpec((NPAD, 1), lambda: (0, 0)),
            pl.BlockSpec((NPAD, 1), lambda: (0, 0)),
            pl.BlockSpec((1, E), lambda: (0, 0), memory_space=pltpu.SMEM),
            pl.BlockSpec((1, E), lambda: (0, 0), memory_space=pltpu.SMEM),
        ],
        out_shape=[
            jax.ShapeDtypeStruct((NPAD, 1), jnp.float32),
            jax.ShapeDtypeStruct((NPAD, 1), jnp.float32),
            jax.ShapeDtypeStruct((1, E), jnp.int32),
            jax.ShapeDtypeStruct((1, E), jnp.int32),
        ],
    )(x, router)


_NW = 32
_RPW = NPAD // _NW


@functools.cache
def _sc_gather_fn():
    mesh = plsc.VectorSubcoreMesh(core_axis_name="c", subcore_axis_name="s")

    @jax.jit
    @functools.partial(
        pl.kernel,
        mesh=mesh,
        out_type=jax.ShapeDtypeStruct((NPAD, D), jnp.float32),
        scratch_types=[
            pltpu.VMEM((_RPW,), jnp.int32),
            pltpu.VMEM((_RPW, D), jnp.float32),
            pltpu.SemaphoreType.DMA,
        ],
    )
    def _sc_gather(x_hbm, idx_hbm, xg_hbm, idx_v, rows_v, sem):
        wid = lax.axis_index("s") * 2 + lax.axis_index("c")
        base = wid * _RPW
        pltpu.sync_copy(idx_hbm.at[pl.ds(base, _RPW)], idx_v)
        pltpu.async_copy(x_hbm.at[idx_v], rows_v, sem).wait()
        pltpu.sync_copy(rows_v, xg_hbm.at[pl.ds(base, _RPW)])

    return _sc_gather


def _mlp_body(nblk_ref, off_ref, xg_ref, g_ref, u_ref, d_ref, rt_ref, rw_ref,
              o_ref):
    e = pl.program_id(0)

    @pl.when(e == 0)
    def _init():
        o_ref[...] = jnp.zeros_like(o_ref)

    n = nblk_ref[0, e]
    off = off_ref[0, e]

    def chunk(j, carry):
        r0 = (off + j) * BLK
        xrows = xg_ref[pl.ds(r0, BLK), :]
        g = lax.dot_general(xrows, g_ref[0], (((1,), (1,)), ((), ())),
                            preferred_element_type=jnp.float32)
        u = lax.dot_general(xrows, u_ref[0], (((1,), (1,)), ((), ())),
                            preferred_element_type=jnp.float32)
        inter = g * lax.logistic(g) * u
        eo = lax.dot_general(inter, d_ref[0], (((1,), (1,)), ((), ())),
                             preferred_element_type=jnp.float32)
        tok = rt_ref[pl.ds(r0, BLK), :]
        w = rw_ref[pl.ds(r0, BLK), :]
        lane_t = lax.broadcasted_iota(jnp.int32, (BLK, T), 1).astype(jnp.float32)
        c = jnp.where(lane_t == tok, w, 0.0)
        o_ref[...] += lax.dot_general(c, eo, (((0,), (0,)), ((), ())),
                                      preferred_element_type=jnp.float32)
        return carry

    lax.fori_loop(0, n, chunk, 0)


@jax.jit
def _mlp(nblk, off, xg, gate_proj, up_proj, down_proj, rt, rw):
    return pl.pallas_call(
        _mlp_body,
        grid=(E,),
        in_specs=[
            pl.BlockSpec((1, E), lambda e: (0, 0), memory_space=pltpu.SMEM),
            pl.BlockSpec((1, E), lambda e: (0, 0), memory_space=pltpu.SMEM),
            pl.BlockSpec((NPAD, D), lambda e: (0, 0)),
            pl.BlockSpec((1, DFF, D), lambda e: (e, 0, 0)),
            pl.BlockSpec((1, DFF, D), lambda e: (e, 0, 0)),
            pl.BlockSpec((1, D, DFF), lambda e: (e, 0, 0)),
            pl.BlockSpec((NPAD, 1), lambda e: (0, 0)),
            pl.BlockSpec((NPAD, 1), lambda e: (0, 0)),
        ],
        out_specs=pl.BlockSpec((T, D), lambda e: (0, 0)),
        out_shape=jax.ShapeDtypeStruct((T, D), jnp.float32),
    )(nblk, off, xg, gate_proj, up_proj, down_proj, rt, rw)


def kernel(hidden_states, router, gate_proj, up_proj, down_proj):
    b, s, d = hidden_states.shape
    x = hidden_states.reshape(-1, d)
    rt, rw, nblk, off = _routing(x, router)
    rt_i = rt.reshape(NPAD).astype(jnp.int32)
    xg = _sc_gather_fn()(x, rt_i)
    out = _mlp(nblk, off, xg, gate_proj, up_proj, down_proj, rt, rw)
    return out.reshape(b, s, d)


# speedup vs baseline: 2.2138x; 2.2138x over previous
import functools

import jax
import jax.numpy as jnp
from jax import lax
from jax.experimental import pallas as pl
from jax.experimental.pallas import tpu as pltpu

B, S, D = 16, 32, 1024
E, TOPK, DFF = 8, 2, 1024
T = B * S
BLK = 128
NR = T * TOPK
NPAD = NR + E * BLK


def _routing_body(x_ref, r_ref, rt_ref, rw_ref, nblk_ref, off_ref):
    x = x_ref[...]
    logits = lax.dot_general(x, r_ref[...], (((1,), (1,)), ((), ())),
                             preferred_element_type=jnp.float32)
    idx = lax.broadcasted_iota(jnp.int32, (T, E), 1)
    m0 = jnp.max(logits, axis=1, keepdims=True)
    i0 = jnp.min(jnp.where(logits == m0, idx, E), axis=1, keepdims=True)
    masked = jnp.where(idx == i0, -jnp.inf, logits)
    m1 = jnp.max(masked, axis=1, keepdims=True)
    i1 = jnp.min(jnp.where(masked == m1, idx, E), axis=1, keepdims=True)
    e1 = jnp.exp(m1 - m0)
    denom = 1.0 + e1
    w0 = 1.0 / denom
    w1 = e1 / denom
    s = w0 + w1
    w0 = w0 / s
    w1 = w1 / s

    tril = (lax.broadcasted_iota(jnp.int32, (T, T), 0)
            > lax.broadcasted_iota(jnp.int32, (T, T), 1)).astype(jnp.float32)
    pos0 = jnp.zeros((T, 1), jnp.float32)
    pos1 = jnp.zeros((T, 1), jnp.float32)
    offblk = jnp.int32(0)
    for e in range(E):
        m0e = (i0 == e)
        m1e = (i1 == e)
        f0 = m0e.astype(jnp.float32)
        f1 = m1e.astype(jnp.float32)
        cnt_t = f0 + f1
        pre = lax.dot_general(tril, cnt_t, (((1,), (0,)), ((), ())),
                              preferred_element_type=jnp.float32)
        count_e = jnp.sum(cnt_t).astype(jnp.int32)
        nb_e = (count_e + BLK - 1) // BLK
        base = (offblk * BLK).astype(jnp.float32)
        pos0 = pos0 + jnp.where(m0e, base + pre, 0.0)
        pos1 = pos1 + jnp.where(m1e, base + pre + f0, 0.0)
        nblk_ref[0, e] = nb_e
        off_ref[0, e] = offblk
        offblk = offblk + nb_e

    lane_r = lax.broadcasted_iota(jnp.int32, (T, NPAD), 1).astype(jnp.float32)
    m0t = jnp.where(lane_r == pos0, 1.0, 0.0)
    m1t = jnp.where(lane_r == pos1, 1.0, 0.0)
    t_col = lax.broadcasted_iota(jnp.int32, (T, 1), 0).astype(jnp.float32)
    rhs0 = jnp.concatenate([t_col, w0], axis=1)
    rhs1 = jnp.concatenate([t_col, w1], axis=1)
    out0 = lax.dot_general(m0t, rhs0, (((0,), (0,)), ((), ())),
                           preferred_element_type=jnp.float32)
    out1 = lax.dot_general(m1t, rhs1, (((0,), (0,)), ((), ())),
                           preferred_element_type=jnp.float32)
    rt_ref[...] = out0[:, 0:1] + out1[:, 0:1]
    rw_ref[...] = out0[:, 1:2] + out1[:, 1:2]


@jax.jit
def _routing(x, router):
    return pl.pallas_call(
        _routing_body,
        in_specs=[
            pl.BlockSpec((T, D), lambda: (0, 0)),
            pl.BlockSpec((E, D), lambda: (0, 0)),
        ],
        out_specs=[
            pl.BlockSpec((NPAD, 1), lambda: (0, 0)),
            pl.BlockSpec((NPAD, 1), lambda: (0, 0)),
            pl.BlockSpec((1, E), lambda: (0, 0), memory_space=pltpu.SMEM),
            pl.BlockSpec((1, E), lambda: (0, 0), memory_space=pltpu.SMEM),
        ],
        out_shape=[
            jax.ShapeDtypeStruct((NPAD, 1), jnp.float32),
            jax.ShapeDtypeStruct((NPAD, 1), jnp.float32),
            jax.ShapeDtypeStruct((1, E), jnp.int32),
            jax.ShapeDtypeStruct((1, E), jnp.int32),
        ],
    )(x, router)


def _mlp_body(nblk_ref, off_ref, x_ref, g_ref, u_ref, d_ref, rt_ref, rw_ref,
              o_ref):
    e = pl.program_id(0)

    @pl.when(e == 0)
    def _init():
        o_ref[...] = jnp.zeros_like(o_ref)

    n = nblk_ref[0, e]
    off = off_ref[0, e]

    def chunk(j, carry):
        r0 = (off + j) * BLK
        tok = rt_ref[pl.ds(r0, BLK), :]
        w = rw_ref[pl.ds(r0, BLK), :]
        lane_t = lax.broadcasted_iota(jnp.int32, (BLK, T), 1).astype(jnp.float32)
        onehot = jnp.where(lane_t == tok, 1.0, 0.0)
        xrows = lax.dot_general(onehot, x_ref[...], (((1,), (0,)), ((), ())),
                                preferred_element_type=jnp.float32)
        g = lax.dot_general(xrows, g_ref[0], (((1,), (1,)), ((), ())),
                            preferred_element_type=jnp.float32)
        u = lax.dot_general(xrows, u_ref[0], (((1,), (1,)), ((), ())),
                            preferred_element_type=jnp.float32)
        inter = g * lax.logistic(g) * u
        eo = lax.dot_general(inter, d_ref[0], (((1,), (1,)), ((), ())),
                             preferred_element_type=jnp.float32)
        c = onehot * w
        o_ref[...] += lax.dot_general(c, eo, (((0,), (0,)), ((), ())),
                                      preferred_element_type=jnp.float32)
        return carry

    lax.fori_loop(0, n, chunk, 0)


@jax.jit
def _mlp(nblk, off, x, gate_proj, up_proj, down_proj, rt, rw):
    return pl.pallas_call(
        _mlp_body,
        grid=(E,),
        in_specs=[
            pl.BlockSpec((1, E), lambda e: (0, 0), memory_space=pltpu.SMEM),
            pl.BlockSpec((1, E), lambda e: (0, 0), memory_space=pltpu.SMEM),
            pl.BlockSpec((T, D), lambda e: (0, 0)),
            pl.BlockSpec((1, DFF, D), lambda e: (e, 0, 0)),
            pl.BlockSpec((1, DFF, D), lambda e: (e, 0, 0)),
            pl.BlockSpec((1, D, DFF), lambda e: (e, 0, 0)),
            pl.BlockSpec((NPAD, 1), lambda e: (0, 0)),
            pl.BlockSpec((NPAD, 1), lambda e: (0, 0)),
        ],
        out_specs=pl.BlockSpec((T, D), lambda e: (0, 0)),
        out_shape=jax.ShapeDtypeStruct((T, D), jnp.float32),
    )(nblk, off, x, gate_proj, up_proj, down_proj, rt, rw)


def kernel(hidden_states, router, gate_proj, up_proj, down_proj):
    b, s, d = hidden_states.shape
    x = hidden_states.reshape(-1, d)
    rt, rw, nblk, off = _routing(x, router)
    out = _mlp(nblk, off, x, gate_proj, up_proj, down_proj, rt, rw)
    return out.reshape(b, s, d)
